# baseline (speedup 1.0000x reference)
import jax
import jax.numpy as jnp
from jax.experimental import pallas as pl
from jax.experimental.pallas import tpu as pltpu

NBR = 5
B_PER_BLOCK = 25
NSPLIT = 4


def _body(x_ref, *rest):
    n_refs = rest[:NSPLIT]
    wb_ref, wl_ref, wr_ref, o_ref = rest[NSPLIT:]
    bb, j, d = x_ref.shape
    jc = j // NSPLIT
    for s in range(NSPLIT):
        n_ref = n_refs[s]
        agg = n_ref[:, :, 0, :] * wb_ref[0, :]
        for k in range(1, NBR):
            agg = agg + n_ref[:, :, k, :] * wb_ref[k, :]
        xb = x_ref[:, s * jc:(s + 1) * jc, :].reshape(bb * jc, d)
        aggb = agg.reshape(bb * jc, d)
        o_ref[:, s * jc:(s + 1) * jc, :] = (
            jnp.dot(xb, wl_ref[...], preferred_element_type=jnp.float32)
            + jnp.dot(aggb, wr_ref[...], preferred_element_type=jnp.float32)
        ).reshape(bb, jc, d)


def kernel(x, neigh_x, w_aggr1, W_l, W_r):
    b, j, d = x.shape
    n_rows = b * j
    wb = jnp.pad(
        jnp.broadcast_to(w_aggr1[0][:, None], (NBR, d)), ((0, 8 - NBR), (0, 0))
    )
    wl_t = W_l.T
    wr_t = W_r.T

    bb = B_PER_BLOCK
    jc = j // NSPLIT
    grid = (b // bb,)
    neigh_specs = [
        pl.BlockSpec(
            (bb, jc, NBR, d),
            lambda i, s=s: (i, s, 0, 0),
        )
        for s in range(NSPLIT)
    ]
    out = pl.pallas_call(
        _body,
        grid=grid,
        in_specs=[
            pl.BlockSpec((bb, j, d), lambda i: (i, 0, 0)),
            *neigh_specs,
            pl.BlockSpec((8, d), lambda i: (0, 0)),
            pl.BlockSpec((d, d), lambda i: (0, 0)),
            pl.BlockSpec((d, d), lambda i: (0, 0)),
        ],
        out_specs=pl.BlockSpec((bb, j, d), lambda i: (i, 0, 0)),
        out_shape=jax.ShapeDtypeStruct((b, j, d), jnp.float32),
        compiler_params=pltpu.CompilerParams(
            dimension_semantics=("arbitrary",),
        ),
    )(x, *([neigh_x] * NSPLIT), wb, wl_t, wr_t)
    return out.reshape(n_rows, d)
